# baseline (speedup 1.0000x reference)
import jax
import jax.numpy as jnp
from jax.experimental import pallas as pl
from jax.experimental.pallas import tpu as pltpu

_EULER_GAMMA = 0.57721566490153286060
_EPS = 1e-13
_LOG2E = 1.4426950408889634
_C2 = 2.0 * _EULER_GAMMA * _LOG2E
_BB = 128


def _duck_body(e_ref, erc_ref, nrc_ref, r_hbm, out_ref, rbuf, acc_ref, sem):
    i = pl.program_id(0)
    nsteps = pl.num_programs(0)
    b, k, _, d = r_hbm.shape
    bb = b // nsteps
    r_flat = r_hbm.reshape(b, k, 2 * d)
    slot = jax.lax.rem(i, 2)
    nxt = jax.lax.rem(i + 1, 2)

    nch = sem.shape[1]
    ch = bb // nch

    def copies_for(block, s):
        return [
            pltpu.make_async_copy(
                r_flat.at[pl.ds(block * bb + c * ch, ch)],
                rbuf.at[s, pl.ds(c * ch, ch)],
                sem.at[s, c],
            )
            for c in range(nch)
        ]

    @pl.when(i == 0)
    def _first():
        for cp in copies_for(0, 0):
            cp.start()

    @pl.when(i + 1 < nsteps)
    def _prefetch():
        for cp in copies_for(i + 1, nxt):
            cp.start()

    for cp in copies_for(i, slot):
        cp.wait()

    el = (e_ref[:, 0, :] * _LOG2E + _C2)[:, None, :]
    er = (e_ref[:, 1, :] * _LOG2E)[:, None, :]
    r = rbuf[slot]
    rl = r[:, :, :d] * _LOG2E + _C2
    rr = r[:, :, d:] * _LOG2E

    e1 = jnp.exp2(-jnp.abs(er - rr))
    e2 = jnp.exp2(-jnp.abs(el - rl))
    m2 = jnp.minimum(er, rr) - jnp.maximum(el, rl)
    den = (1.0 + e1) * (1.0 + e2)
    sp_i = jnp.log(den + jnp.exp2(m2)) - jnp.log(den)

    x2 = rr - rl
    sp_r = jnp.log(1.0 + jnp.exp2(x2))

    maskf = ((nrc_ref[...] >= 1) & (erc_ref[...] >= 1)).astype(jnp.float32)
    mask3 = maskf[:, :, None] > 0.5
    pe_r = jnp.where(mask3, sp_r + _EPS, 1.0)
    pe_i = jnp.where(mask3, sp_i + _EPS, 1.0)
    pe_r = pe_r.reshape(pe_r.shape[0] // 2, 2, k, d)
    pe_i = pe_i.reshape(pe_i.shape[0] // 2, 2, k, d)
    pr = pe_r[:, 0] * pe_r[:, 1]
    pi = pe_i[:, 0] * pe_i[:, 1]
    part = jnp.sum(jnp.log(pr) - jnp.log(pi), axis=(0, 1))

    @pl.when(i == 0)
    def _init():
        acc_ref[...] = jnp.zeros_like(acc_ref)

    acc_ref[...] += part[None, :]

    @pl.when(i == nsteps - 1)
    def _finish():
        out_ref[...] = (jnp.sum(acc_ref[...]) / (nsteps * bb * k)).reshape(1, 1)


def kernel(entity_box, rel_box, entity_rel_counts, neighbor_rel_counts):
    b, k, _, d = rel_box.shape
    erc = entity_rel_counts.reshape(b, 1)
    out = pl.pallas_call(
        _duck_body,
        grid=(b // _BB,),
        in_specs=[
            pl.BlockSpec((_BB, 2, d), lambda i: (i, 0, 0)),
            pl.BlockSpec((_BB, 1), lambda i: (i, 0)),
            pl.BlockSpec((_BB, k), lambda i: (i, 0)),
            pl.BlockSpec(memory_space=pltpu.MemorySpace.HBM),
        ],
        out_specs=pl.BlockSpec((1, 1), lambda i: (0, 0)),
        out_shape=jax.ShapeDtypeStruct((1, 1), jnp.float32),
        scratch_shapes=[
            pltpu.VMEM((2, _BB, k, 2 * d), jnp.float32),
            pltpu.VMEM((1, d), jnp.float32),
            pltpu.SemaphoreType.DMA((2, 4)),
        ],
    )(entity_box, erc, neighbor_rel_counts, rel_box)
    return out.reshape(())
